# baseline (speedup 1.0000x reference)
import functools

import jax
import jax.numpy as jnp
from jax import lax
from jax.experimental import pallas as pl
from jax.experimental.pallas import tpu as pltpu
from jax.experimental.pallas import tpu_sc as plsc

VOCAB = 100000
SEQ_LEN = 2048
EMBED = 128
BATCH = 4

NC = 2
NS = 16
NW = NC * NS
B_PER_W = (BATCH * SEQ_LEN) // NW
CH = 32
NCH = B_PER_W // CH
W_PER_B = SEQ_LEN // B_PER_W
LANES = 16
SCALE = 11.31370849898476


def _sc_embed(idx, token_table, pos_table):
  mesh = plsc.VectorSubcoreMesh(core_axis_name="c", subcore_axis_name="s")

  @functools.partial(
      pl.kernel,
      mesh=mesh,
      out_type=jax.ShapeDtypeStruct((BATCH, SEQ_LEN, EMBED), jnp.float32),
      scratch_types=[
          pltpu.VMEM((B_PER_W,), jnp.int32),
          pltpu.VMEM((B_PER_W, EMBED), jnp.float32),
          pltpu.VMEM((B_PER_W, EMBED), jnp.float32),
          pltpu.SemaphoreType.DMA,
          pltpu.SemaphoreType.DMA((NCH,)),
          pltpu.SemaphoreType.DMA((NCH,)),
          pltpu.SemaphoreType.DMA,
      ],
  )
  def k(idx_hbm, tok_hbm, pos_hbm, out_hbm, idx_v, tok_v, pos_v,
        isem, gsem, psem, wsem):
    wid = lax.axis_index("s") * NC + lax.axis_index("c")
    b = wid // W_PER_B
    s0 = (wid % W_PER_B) * B_PER_W
    idx_copy = pltpu.async_copy(
        idx_hbm.at[b, pl.ds(s0, B_PER_W)], idx_v, isem)

    def pos_chunk(c):
      return pltpu.async_copy(
          pos_hbm.at[pl.ds(s0 + c * CH, CH)],
          pos_v.at[pl.ds(c * CH, CH)],
          psem.at[c],
      )

    pos_copies = [pos_chunk(0), pos_chunk(1)]
    idx_copy.wait()
    gathers = [
        pltpu.async_copy(
            tok_hbm.at[idx_v.at[pl.ds(c * CH, CH)]],
            tok_v.at[pl.ds(c * CH, CH)],
            gsem.at[c],
        ) for c in range(NCH)
    ]
    for c in range(2, NCH):
      pos_copies.append(pos_chunk(c))

    writes = []
    for c in range(NCH):
      gathers[c].wait()
      pos_copies[c].wait()

      def row(r, carry):
        for j in range(EMBED // LANES):
          sl = (r, pl.ds(j * LANES, LANES))
          plsc.addupdate(pos_v.at[sl], tok_v[sl] * SCALE)
        return carry

      lax.fori_loop(c * CH, (c + 1) * CH, row, 0, unroll=2)
      writes.append(
          pltpu.async_copy(
              pos_v.at[pl.ds(c * CH, CH)],
              out_hbm.at[b].at[pl.ds(s0 + c * CH, CH)],
              wsem,
          ))
    for w in writes:
      w.wait()

  return k(idx, token_table, pos_table)


def kernel(inputs, token_table, pos_table):
  return _sc_embed(inputs.astype(jnp.int32), token_table, pos_table)


# speedup vs baseline: 1.0659x; 1.0659x over previous
import functools

import jax
import jax.numpy as jnp
from jax import lax
from jax.experimental import pallas as pl
from jax.experimental.pallas import tpu as pltpu
from jax.experimental.pallas import tpu_sc as plsc

VOCAB = 100000
SEQ_LEN = 2048
EMBED = 128
BATCH = 4

NC = 2
NS = 16
NW = NC * NS
B_PER_W = (BATCH * SEQ_LEN) // NW
CH = 64
NCH = B_PER_W // CH
W_PER_B = SEQ_LEN // B_PER_W
LANES = 16
SCALE = 11.31370849898476


def _sc_embed(idx, token_table, pos_table):
  mesh = plsc.VectorSubcoreMesh(core_axis_name="c", subcore_axis_name="s")

  @functools.partial(
      pl.kernel,
      mesh=mesh,
      out_type=jax.ShapeDtypeStruct((BATCH, SEQ_LEN, EMBED), jnp.float32),
      scratch_types=[
          pltpu.VMEM((NCH, CH), jnp.int32),
          pltpu.VMEM((B_PER_W, EMBED), jnp.float32),
          pltpu.VMEM((B_PER_W, EMBED), jnp.float32),
          pltpu.VMEM_SHARED((SEQ_LEN, EMBED), jnp.float32),
          pltpu.SemaphoreType.DMA((NCH,)),
          pltpu.SemaphoreType.DMA((NCH,)),
          pltpu.SemaphoreType.DMA((NCH,)),
          pltpu.SemaphoreType.DMA((NCH,)),
          pltpu.SemaphoreType.DMA,
      ],
  )
  def k(idx_hbm, tok_hbm, pos_hbm, out_hbm, idx_v, tok_v, pos_v, spos,
        isem, gsem, psem, wsem, ssem):
    sid = lax.axis_index("s")
    wid = sid * NC + lax.axis_index("c")
    b = wid // W_PER_B
    s0 = (wid % W_PER_B) * B_PER_W
    srows = SEQ_LEN // NS
    stage = pltpu.async_copy(
        pos_hbm.at[pl.ds(sid * srows, srows)],
        spos.at[pl.ds(sid * srows, srows)],
        ssem,
    )
    idx_copies = [
        pltpu.async_copy(idx_hbm.at[b, pl.ds(s0 + c * CH, CH)],
                         idx_v.at[c], isem.at[c])
        for c in range(NCH)
    ]
    gathers = []
    for c in range(NCH):
      idx_copies[c].wait()
      gathers.append(
          pltpu.async_copy(
              tok_hbm.at[idx_v.at[c]],
              tok_v.at[pl.ds(c * CH, CH)],
              gsem.at[c],
          ))
    stage.wait()
    plsc.subcore_barrier()
    pos_copies = [
        pltpu.async_copy(
            spos.at[pl.ds(s0 + c * CH, CH)],
            pos_v.at[pl.ds(c * CH, CH)],
            psem.at[c],
        ) for c in range(NCH)
    ]

    writes = []
    for c in range(NCH):
      gathers[c].wait()
      pos_copies[c].wait()

      def row(r, carry):
        for j in range(EMBED // LANES):
          sl = (r, pl.ds(j * LANES, LANES))
          plsc.addupdate(pos_v.at[sl], tok_v[sl] * SCALE)
        return carry

      lax.fori_loop(c * CH, (c + 1) * CH, row, 0, unroll=2)
      writes.append(
          pltpu.async_copy(
              pos_v.at[pl.ds(c * CH, CH)],
              out_hbm.at[b].at[pl.ds(s0 + c * CH, CH)],
              wsem.at[c],
          ))
    for w in writes:
      w.wait()

  return k(idx, token_table, pos_table)


def kernel(inputs, token_table, pos_table):
  return _sc_embed(inputs.astype(jnp.int32), token_table, pos_table)
